# baseline (speedup 1.0000x reference)
import functools
import math

import jax
import jax.numpy as jnp
import numpy as np
from jax import lax
from jax.experimental import pallas as pl
from jax.experimental.pallas import tpu as pltpu
from jax.experimental.pallas import tpu_sc as plsc

DIM = 1024
MAX_T = 8192
BASE = 10000.0

NC, NS, L = 2, 16, 16
NW = NC * NS
N_TOK = 4 * 4096
TOK_PER_W = N_TOK // NW
K = 16
CHUNKS = TOK_PER_W // K


def _make_pe_np():
    pos = np.arange(MAX_T, dtype=np.float64)[:, None]
    div = np.exp(np.arange(0, DIM, 2, dtype=np.float64) * -(math.log(BASE) / DIM))
    pe = np.zeros((MAX_T, DIM), dtype=np.float32)
    pe[:, 0::2] = np.sin(pos * div).astype(np.float32)
    pe[:, 1::2] = np.cos(pos * div).astype(np.float32)
    return pe


_PE = _make_pe_np()

_mesh = plsc.VectorSubcoreMesh(core_axis_name="c", subcore_axis_name="s")


@functools.partial(
    pl.kernel,
    out_type=jax.ShapeDtypeStruct((N_TOK, DIM), jnp.float32),
    mesh=_mesh,
    scratch_types=[
        pltpu.VMEM((K,), jnp.int32),
        pltpu.VMEM((K, DIM), jnp.float32),
        pltpu.VMEM((K, DIM), jnp.float32),
        pltpu.SemaphoreType.DMA,
        pltpu.SemaphoreType.DMA,
    ],
)
def _sc_pe_add(pe_hbm, x_hbm, tid_hbm, out_hbm, idx_v, rows_v, x_v, sem_g, sem_x):
    wid = lax.axis_index("s") * NC + lax.axis_index("c")
    base = wid * TOK_PER_W

    def chunk(c, carry):
        off = base + c * K
        pltpu.sync_copy(tid_hbm.at[pl.ds(off, K)], idx_v)
        g = pltpu.async_copy(pe_hbm.at[idx_v], rows_v, sem_g)
        xc = pltpu.async_copy(x_hbm.at[pl.ds(off, K)], x_v, sem_x)
        g.wait()
        xc.wait()
        for r in range(K):
            def addcol(j, acc):
                sl = pl.ds(j * L, L)
                x_v[r, sl] = x_v[r, sl] + rows_v[r, sl]
                return acc

            lax.fori_loop(0, DIM // L, addcol, 0, unroll=4)
        pltpu.sync_copy(x_v, out_hbm.at[pl.ds(off, K)])
        return carry

    lax.fori_loop(0, CHUNKS, chunk, 0)


@jax.jit
def kernel(x, time_ids):
    b, s, dim = x.shape
    xf = x.reshape(N_TOK, dim)
    tf = time_ids.reshape(N_TOK).astype(jnp.int32)
    pe = jnp.asarray(_PE)
    out = _sc_pe_add(pe, xf, tf)
    return out.reshape(b, s, dim)


# speedup vs baseline: 1.2523x; 1.2523x over previous
import functools
import math

import jax
import jax.numpy as jnp
import numpy as np
from jax import lax
from jax.experimental import pallas as pl
from jax.experimental.pallas import tpu as pltpu
from jax.experimental.pallas import tpu_sc as plsc

DIM = 1024
MAX_T = 8192
BASE = 10000.0

NC, NS, L = 2, 16, 16
NW = NC * NS
N_TOK = 4 * 4096
TOK_PER_W = N_TOK // NW
K = 16
CHUNKS = TOK_PER_W // K
NPAIR = CHUNKS // 2


def _make_pe_np():
    pos = np.arange(MAX_T, dtype=np.float64)[:, None]
    div = np.exp(np.arange(0, DIM, 2, dtype=np.float64) * -(math.log(BASE) / DIM))
    pe = np.zeros((MAX_T, DIM), dtype=np.float32)
    pe[:, 0::2] = np.sin(pos * div).astype(np.float32)
    pe[:, 1::2] = np.cos(pos * div).astype(np.float32)
    return pe


_PE = _make_pe_np()

_mesh = plsc.VectorSubcoreMesh(core_axis_name="c", subcore_axis_name="s")


@functools.partial(
    pl.kernel,
    out_type=jax.ShapeDtypeStruct((N_TOK, DIM), jnp.float32),
    mesh=_mesh,
    scratch_types=[
        pltpu.VMEM((TOK_PER_W,), jnp.int32),
        pltpu.VMEM((K, DIM), jnp.float32),
        pltpu.VMEM((K, DIM), jnp.float32),
        pltpu.VMEM((K, DIM), jnp.float32),
        pltpu.VMEM((K, DIM), jnp.float32),
        pltpu.SemaphoreType.DMA,
        pltpu.SemaphoreType.DMA,
        pltpu.SemaphoreType.DMA,
        pltpu.SemaphoreType.DMA,
        pltpu.SemaphoreType.DMA,
        pltpu.SemaphoreType.DMA,
    ],
)
def _sc_pe_add(
    pe_hbm, x_hbm, tid_hbm, out_hbm,
    idx_all, rows0, rows1, xv0, xv1,
    sem_g0, sem_g1, sem_x0, sem_x1, sem_o0, sem_o1,
):
    wid = lax.axis_index("s") * NC + lax.axis_index("c")
    base = wid * TOK_PER_W
    pltpu.sync_copy(tid_hbm.at[pl.ds(base, TOK_PER_W)], idx_all)

    def start_in(c, rows_v, x_v, sem_g, sem_x):
        idx = idx_all.at[pl.ds(c * K, K)]
        pltpu.async_copy(pe_hbm.at[idx], rows_v, sem_g)
        pltpu.async_copy(x_hbm.at[pl.ds(base + c * K, K)], x_v, sem_x)

    def wait_in(rows_v, x_v, sem_g, sem_x):
        pltpu.make_async_copy(pe_hbm.at[pl.ds(0, K)], rows_v, sem_g).wait()
        pltpu.make_async_copy(x_hbm.at[pl.ds(0, K)], x_v, sem_x).wait()

    def start_out(c, x_v, sem_o):
        pltpu.async_copy(x_v, out_hbm.at[pl.ds(base + c * K, K)], sem_o)

    def wait_out(x_v, sem_o):
        pltpu.make_async_copy(x_v, out_hbm.at[pl.ds(0, K)], sem_o).wait()

    def add_chunk(rows_v, x_v):
        for r in range(K):
            def addcol(j, acc):
                sl = pl.ds(j * L, L)
                x_v[r, sl] = x_v[r, sl] + rows_v[r, sl]
                return acc

            lax.fori_loop(0, DIM // L, addcol, 0, unroll=4)

    start_in(0, rows0, xv0, sem_g0, sem_x0)

    def pair(i, carry):
        c0 = 2 * i
        c1 = c0 + 1

        @pl.when(i > 0)
        def _():
            wait_out(xv1, sem_o1)

        start_in(c1, rows1, xv1, sem_g1, sem_x1)
        wait_in(rows0, xv0, sem_g0, sem_x0)
        add_chunk(rows0, xv0)
        start_out(c0, xv0, sem_o0)

        wait_in(rows1, xv1, sem_g1, sem_x1)
        add_chunk(rows1, xv1)
        start_out(c1, xv1, sem_o1)

        @pl.when(i + 1 < NPAIR)
        def _():
            wait_out(xv0, sem_o0)
            start_in(c0 + 2, rows0, xv0, sem_g0, sem_x0)

        return carry

    lax.fori_loop(0, NPAIR, pair, 0)
    wait_out(xv0, sem_o0)
    wait_out(xv1, sem_o1)


@jax.jit
def kernel(x, time_ids):
    b, s, dim = x.shape
    xf = x.reshape(N_TOK, dim)
    tf = time_ids.reshape(N_TOK).astype(jnp.int32)
    pe = jnp.asarray(_PE)
    out = _sc_pe_add(pe, xf, tf)
    return out.reshape(b, s, dim)
